# initial kernel scaffold (unmeasured)
import jax
import jax.numpy as jnp
from jax import lax
from jax.experimental import pallas as pl
from jax.experimental.pallas import tpu as pltpu

N_DEV = 4
SQ = 512
SKV = 2048
HQ = 8
HKV = 2
GROUP = HQ // HKV
DH = 128
D = 1024
SCALE = 0.08838834764831843
BF16 = jnp.bfloat16
F32 = jnp.float32


def kernel(x, Wq, Wo, K_ext, V_ext):
    def body(x_ref, wq_ref, wo_ref, k_ref, v_ref, out_ref,
             o_send, ml_send, o_recv, ml_recv,
             o_send_sem, o_recv_sem, ml_send_sem, ml_recv_sem):
        my_pos = lax.axis_index("i")
        peer0 = 3 - my_pos
        peer1 = my_pos + 1 - 2 * (my_pos % 2)

        barrier_sem = pltpu.get_barrier_semaphore()
        for nbr in (peer0, peer1):
            pl.semaphore_signal(barrier_sem, inc=1, device_id=(nbr,),
                                device_id_type=pl.DeviceIdType.MESH)
        pl.semaphore_wait(barrier_sem, 2)

        q = jnp.dot(x_ref[0].astype(BF16), wq_ref[...].astype(BF16),
                    preferred_element_type=F32) * SCALE

        o_run, m_run, l_run = [], [], []
        for h in range(HQ):
            kvh = h // GROUP
            qh = q[:, h * DH:(h + 1) * DH].astype(BF16)
            kh = k_ref[0, :, kvh, :].astype(BF16)
            vh = v_ref[0, :, kvh, :].astype(BF16)
            s = lax.dot_general(qh, kh, (((1,), (1,)), ((), ())),
                                preferred_element_type=F32)
            m = jnp.max(s, axis=1, keepdims=True)
            p = jnp.exp(s - m)
            l = jnp.sum(p, axis=1, keepdims=True)
            o = jnp.dot(p.astype(BF16), vh,
                        preferred_element_type=F32)
            o_run.append(o)
            m_run.append(m)
            l_run.append(l)

        def write_send(slot):
            for h in range(HQ):
                o_send[slot, h * SQ:(h + 1) * SQ, :] = o_run[h].astype(BF16)
            ml = jnp.concatenate(m_run + l_run, axis=1)
            ml_send[slot, :, 0:2 * HQ] = ml

        def merge(slot):
            ml_b = ml_recv[slot, :, 0:2 * HQ]
            for h in range(HQ):
                m_b = ml_b[:, h:h + 1]
                l_b = ml_b[:, HQ + h:HQ + h + 1]
                o_b = o_recv[slot, h * SQ:(h + 1) * SQ, :].astype(F32)
                m_n = jnp.maximum(m_run[h], m_b)
                w_a = jnp.exp(m_run[h] - m_n)
                w_b = jnp.exp(m_b - m_n)
                o_run[h] = o_run[h] * w_a + o_b * w_b
                l_run[h] = l_run[h] * w_a + l_b * w_b
                m_run[h] = m_n

        for r, peer in enumerate((peer0, peer1)):
            write_send(r)
            o_rdma = pltpu.make_async_remote_copy(
                src_ref=o_send.at[r], dst_ref=o_recv.at[r],
                send_sem=o_send_sem.at[r], recv_sem=o_recv_sem.at[r],
                device_id=(peer,), device_id_type=pl.DeviceIdType.MESH)
            ml_rdma = pltpu.make_async_remote_copy(
                src_ref=ml_send.at[r], dst_ref=ml_recv.at[r],
                send_sem=ml_send_sem.at[r], recv_sem=ml_recv_sem.at[r],
                device_id=(peer,), device_id_type=pl.DeviceIdType.MESH)
            o_rdma.start()
            ml_rdma.start()
            o_rdma.wait()
            ml_rdma.wait()
            merge(r)

        att = jnp.concatenate([o_run[h] / l_run[h] for h in range(HQ)],
                              axis=1)
        out_ref[0] = jnp.dot(att.astype(BF16), wo_ref[...].astype(BF16),
                             preferred_element_type=F32)

    return pl.pallas_call(
        body,
        out_shape=jax.ShapeDtypeStruct((1, SQ, D), F32),
        in_specs=[pl.BlockSpec(memory_space=pltpu.VMEM)] * 5,
        out_specs=pl.BlockSpec(memory_space=pltpu.VMEM),
        scratch_shapes=[
            pltpu.VMEM((2, HQ * SQ, DH), BF16),
            pltpu.VMEM((2, SQ, 128), F32),
            pltpu.VMEM((2, HQ * SQ, DH), BF16),
            pltpu.VMEM((2, SQ, 128), F32),
            pltpu.SemaphoreType.DMA((2,)),
            pltpu.SemaphoreType.DMA((2,)),
            pltpu.SemaphoreType.DMA((2,)),
            pltpu.SemaphoreType.DMA((2,)),
        ],
        compiler_params=pltpu.CompilerParams(collective_id=0),
    )(x, Wq, Wo, K_ext, V_ext)


# baseline (device time: 67136 ns/iter reference)
import jax
import jax.numpy as jnp
from jax import lax
from jax.experimental import pallas as pl
from jax.experimental.pallas import tpu as pltpu

N_DEV = 4
SQ = 512
SKV = 2048
HQ = 8
HKV = 2
GROUP = HQ // HKV
DH = 128
D = 1024
SCALE = 0.08838834764831843
BF16 = jnp.bfloat16
F32 = jnp.float32


def kernel(x, Wq, Wo, K_ext, V_ext):
    def body(x_ref, wq_ref, wo_ref, k_ref, v_ref, out_ref,
             o_send, ml_send, o_recv, ml_recv,
             o_send_sem, o_recv_sem, ml_send_sem, ml_recv_sem):
        my_pos = lax.axis_index("i")
        peer0 = 3 - my_pos
        peer1 = my_pos + 1 - 2 * (my_pos % 2)

        barrier_sem = pltpu.get_barrier_semaphore()
        for nbr in (peer0, peer1):
            pl.semaphore_signal(barrier_sem, inc=1, device_id=(nbr,),
                                device_id_type=pl.DeviceIdType.MESH)
        pl.semaphore_wait(barrier_sem, 2)

        q = jnp.dot(x_ref[0].astype(BF16), wq_ref[...].astype(BF16),
                    preferred_element_type=F32) * SCALE

        o_run, m_run, l_run = [], [], []
        for h in range(HQ):
            kvh = h // GROUP
            qh = q[:, h * DH:(h + 1) * DH].astype(BF16)
            kh = k_ref[0, :, kvh, :].astype(BF16)
            vh = v_ref[0, :, kvh, :].astype(BF16)
            s = lax.dot_general(qh, kh, (((1,), (1,)), ((), ())),
                                preferred_element_type=F32)
            m = jnp.max(s, axis=1, keepdims=True)
            p = jnp.exp(s - m)
            l = jnp.sum(p, axis=1, keepdims=True)
            o = jnp.dot(p.astype(BF16), vh,
                        preferred_element_type=F32)
            o_run.append(o)
            m_run.append(m)
            l_run.append(l)

        def write_send(slot):
            for h in range(HQ):
                o_send[slot, h * SQ:(h + 1) * SQ, :] = o_run[h].astype(BF16)
            ml = jnp.concatenate(m_run + l_run, axis=1)
            ml_send[slot, :, 0:2 * HQ] = ml

        def merge(slot):
            ml_b = ml_recv[slot, :, 0:2 * HQ]
            for h in range(HQ):
                m_b = ml_b[:, h:h + 1]
                l_b = ml_b[:, HQ + h:HQ + h + 1]
                o_b = o_recv[slot, h * SQ:(h + 1) * SQ, :].astype(F32)
                m_n = jnp.maximum(m_run[h], m_b)
                w_a = jnp.exp(m_run[h] - m_n)
                w_b = jnp.exp(m_b - m_n)
                o_run[h] = o_run[h] * w_a + o_b * w_b
                l_run[h] = l_run[h] * w_a + l_b * w_b
                m_run[h] = m_n

        for r, peer in enumerate((peer0, peer1)):
            write_send(r)
            o_rdma = pltpu.make_async_remote_copy(
                src_ref=o_send.at[r], dst_ref=o_recv.at[r],
                send_sem=o_send_sem.at[r], recv_sem=o_recv_sem.at[r],
                device_id=(peer,), device_id_type=pl.DeviceIdType.MESH)
            ml_rdma = pltpu.make_async_remote_copy(
                src_ref=ml_send.at[r], dst_ref=ml_recv.at[r],
                send_sem=ml_send_sem.at[r], recv_sem=ml_recv_sem.at[r],
                device_id=(peer,), device_id_type=pl.DeviceIdType.MESH)
            o_rdma.start()
            ml_rdma.start()
            o_rdma.wait()
            ml_rdma.wait()
            merge(r)

        att = jnp.concatenate([o_run[h] / l_run[h] for h in range(HQ)],
                              axis=1)
        out_ref[0] = jnp.dot(att.astype(BF16), wo_ref[...].astype(BF16),
                             preferred_element_type=F32)

    return pl.pallas_call(
        body,
        out_shape=jax.ShapeDtypeStruct((1, SQ, D), F32),
        in_specs=[pl.BlockSpec(memory_space=pltpu.VMEM)] * 5,
        out_specs=pl.BlockSpec(memory_space=pltpu.VMEM),
        scratch_shapes=[
            pltpu.VMEM((2, HQ * SQ, DH), BF16),
            pltpu.VMEM((2, SQ, 128), F32),
            pltpu.VMEM((2, HQ * SQ, DH), BF16),
            pltpu.VMEM((2, SQ, 128), F32),
            pltpu.SemaphoreType.DMA((2,)),
            pltpu.SemaphoreType.DMA((2,)),
            pltpu.SemaphoreType.DMA((2,)),
            pltpu.SemaphoreType.DMA((2,)),
        ],
        compiler_params=pltpu.CompilerParams(
            collective_id=0, vmem_limit_bytes=100 * 1024 * 1024),
    )(x, Wq, Wo, K_ext, V_ext)
